# Initial kernel scaffold; baseline (speedup 1.0000x reference)
import jax
import jax.numpy as jnp
from jax.experimental import pallas as pl


def kernel(x, edge_index1, edge_index2, W_l1, W_r1, b1, W_l2, W_r2, b2):
    raise NotImplementedError("write your pallas kernel here")


# speedup vs baseline: 5.8449x; 5.8449x over previous
import functools

import jax
import jax.numpy as jnp
from jax import lax
from jax.experimental import pallas as pl
from jax.experimental.pallas import tpu as pltpu
from jax.experimental.pallas import tpu_sc as plsc

_N = 10000
_E = 320000
_D = 128
_NC = 2
_NS = 16
_NW = _NC * _NS
_EPW = _E // _NW
_C = 80
_NCH = _EPW // _C
_NPAD = 10240
_RPT = _NPAD // _NS
_RC = 80


def _sc_agg(x_t, edges):
    mesh = plsc.VectorSubcoreMesh(core_axis_name="c", subcore_axis_name="s")

    @functools.partial(
        pl.kernel,
        out_type=(
            jax.ShapeDtypeStruct((_NC, _NPAD, _D), jnp.float32),
            jax.ShapeDtypeStruct((_NC, _NPAD), jnp.float32),
        ),
        mesh=mesh,
        compiler_params=pltpu.CompilerParams(needs_layout_passes=False),
        scratch_types=[
            pltpu.VMEM((_C,), jnp.int32),
            pltpu.VMEM((_C,), jnp.int32),
            pltpu.VMEM((_C,), jnp.int32),
            pltpu.VMEM((_C, _D), jnp.float32),
            pltpu.VMEM((_RC, _D), jnp.float32),
            pltpu.VMEM((_NPAD,), jnp.int32),
            pltpu.VMEM((_NPAD // 2,), jnp.int32),
            pltpu.VMEM((_NS // 2, _RPT), jnp.int32),
            pltpu.VMEM((_RPT,), jnp.float32),
            pltpu.VMEM_SHARED((_NPAD, _D), jnp.float32),
            pltpu.VMEM_SHARED((_NS // 2, _NPAD // 2), jnp.int32),
            pltpu.SemaphoreType.DMA,
        ],
    )
    def agg(x_hbm, edge_hbm, sum_hbm, cnt_hbm,
            pi_v, si_v, di_v, rows_v, zr_v, cnt_v, cpk_v, cbuf_v, red_v,
            acc_sh, csh, sem):
        c = lax.axis_index("c")
        s = lax.axis_index("s")
        wid = s * _NC + c

        zeros16 = jnp.zeros((16,), jnp.float32)
        zeros16i = jnp.zeros((16,), jnp.int32)
        ones16i = jnp.ones((16,), jnp.int32)

        def fill_zr(i, carry):
            for k in range(_D // 16):
                zr_v[i, pl.ds(k * 16, 16)] = zeros16
            return carry

        lax.fori_loop(0, _RC, fill_zr, 0)

        def fill_cnt(i, carry):
            cnt_v[pl.ds(i * 16, 16)] = zeros16i
            return carry

        lax.fori_loop(0, _NPAD // 16, fill_cnt, 0)

        base = s * _RPT
        for i in range(_RPT // _RC):
            pltpu.sync_copy(zr_v, acc_sh.at[pl.ds(base + i * _RC, _RC)])
        plsc.subcore_barrier()

        def body(j, carry):
            pltpu.sync_copy(edge_hbm.at[wid, j], pi_v)
            for t in range(_C // 16):
                w = pi_v[pl.ds(t * 16, 16)]
                si_v[pl.ds(t * 16, 16)] = jnp.bitwise_and(w, 0xFFFF)
                d16 = lax.shift_right_logical(w, 16)
                di_v[pl.ds(t * 16, 16)] = d16
                plsc.addupdate_scatter(cnt_v, [d16], ones16i)
            pltpu.async_copy(x_hbm.at[si_v], rows_v, sem).wait()
            pltpu.sync_copy(rows_v, acc_sh.at[di_v], add=True)
            return carry

        lax.fori_loop(0, _NCH, body, 0)

        half = _NPAD // 2

        def packc(g, carry):
            a = cnt_v[pl.ds(g * 16, 16)]
            bq = cnt_v[pl.ds(g * 16 + half, 16)]
            cpk_v[pl.ds(g * 16, 16)] = jnp.bitwise_or(
                a, jnp.left_shift(bq, 16))
            return carry

        lax.fori_loop(0, half // 16, packc, 0)

        colb = (s % (_NS // 2)) * _RPT
        take_lo = s < (_NS // 2)
        nhalf = _NS // 2

        for rnd in range(2):
            @pl.when((s >= rnd * nhalf) & (s < (rnd + 1) * nhalf))
            def _():
                pltpu.sync_copy(cpk_v, csh.at[s % nhalf])

            plsc.subcore_barrier()
            for r in range(nhalf):
                pltpu.sync_copy(csh.at[r, pl.ds(colb, _RPT)],
                                cbuf_v.at[r])

            def red(g, carry):
                acc16 = zeros16i
                for r in range(nhalf):
                    w = cbuf_v[r, pl.ds(g * 16, 16)]
                    part = jnp.where(take_lo,
                                     jnp.bitwise_and(w, 0xFFFF),
                                     lax.shift_right_logical(w, 16))
                    acc16 = acc16 + part
                if rnd == 0:
                    red_v[pl.ds(g * 16, 16)] = acc16.astype(jnp.float32)
                else:
                    red_v[pl.ds(g * 16, 16)] = (
                        red_v[pl.ds(g * 16, 16)]
                        + acc16.astype(jnp.float32))
                return carry

            lax.fori_loop(0, _RPT // 16, red, 0)
            plsc.subcore_barrier()

        pltpu.sync_copy(red_v, cnt_hbm.at[c, pl.ds(base, _RPT)])

        for i in range(_RPT // _RC):
            off = base + i * _RC
            pltpu.sync_copy(acc_sh.at[pl.ds(off, _RC)], rows_v)
            pltpu.sync_copy(rows_v, sum_hbm.at[c, pl.ds(off, _RC)])

    return agg(x_t, edges)


def _dense(psum, pcnt, x_t, w_l, w_r, b, relu, logsm):
    br = 400

    def body(p_ref, c_ref, x_ref, wl_ref, wr_ref, b_ref, o_ref):
        p = p_ref[0] + p_ref[1]
        cnt = c_ref[0] + c_ref[1]
        mean = p / jnp.maximum(cnt, 1.0)
        out = (
            jnp.dot(mean, wl_ref[...], preferred_element_type=jnp.float32,
                    precision=lax.Precision.HIGHEST)
            + jnp.dot(x_ref[...], wr_ref[...],
                      preferred_element_type=jnp.float32,
                      precision=lax.Precision.HIGHEST)
            + b_ref[...]
        )
        if relu:
            out = jnp.maximum(out, 0.0)
        if logsm:
            m = jnp.max(out, axis=1, keepdims=True)
            out = out - m - jnp.log(
                jnp.sum(jnp.exp(out - m), axis=1, keepdims=True))
        o_ref[...] = out

    return pl.pallas_call(
        body,
        grid=(_N // br,),
        in_specs=[
            pl.BlockSpec((2, br, _D), lambda i: (0, i, 0)),
            pl.BlockSpec((2, br, 1), lambda i: (0, i, 0)),
            pl.BlockSpec((br, _D), lambda i: (i, 0)),
            pl.BlockSpec((_D, _D), lambda i: (0, 0)),
            pl.BlockSpec((_D, _D), lambda i: (0, 0)),
            pl.BlockSpec((1, _D), lambda i: (0, 0)),
        ],
        out_specs=pl.BlockSpec((br, _D), lambda i: (i, 0)),
        out_shape=jax.ShapeDtypeStruct((_N, _D), jnp.float32),
    )(psum, pcnt, x_t, w_l, w_r, b)


def _pack_edges(edge_index):
    packed = jnp.bitwise_or(edge_index[0],
                            jnp.left_shift(edge_index[1], 16))
    return packed.reshape(_NW, _NCH, _C)


def kernel(x, edge_index1, edge_index2, W_l1, W_r1, b1, W_l2, W_r2, b2):
    e1 = _pack_edges(edge_index1)
    e2 = _pack_edges(edge_index2)

    s1, c1 = _sc_agg(x, e1)
    h = _dense(s1, c1.reshape(_NC, _NPAD, 1), x, W_l1, W_r1,
               b1.reshape(1, _D), relu=True, logsm=False)
    s2, c2 = _sc_agg(h, e2)
    return _dense(s2, c2.reshape(_NC, _NPAD, 1), h, W_l2, W_r2,
                  b2.reshape(1, _D), relu=False, logsm=True)


# speedup vs baseline: 7.4834x; 1.2803x over previous
import functools

import jax
import jax.numpy as jnp
from jax import lax
from jax.experimental import pallas as pl
from jax.experimental.pallas import tpu as pltpu
from jax.experimental.pallas import tpu_sc as plsc

_N = 10000
_E = 320000
_D = 128
_NC = 2
_NS = 16
_NW = _NC * _NS
_EPW = _E // _NW
_RC0 = 80
_C = 80
_NPAIR = 63
_NPAD = 10240
_ACCR = _NPAD + _RC0
_RPT = _NPAD // _NS
_RC = 80
_TRASH = 10200


def _sc_agg(x_t, edges):
    mesh = plsc.VectorSubcoreMesh(core_axis_name="c", subcore_axis_name="s")

    @functools.partial(
        pl.kernel,
        out_type=jax.ShapeDtypeStruct((_NC, _NPAD + _RC, _D), jnp.float32),
        mesh=mesh,
        compiler_params=pltpu.CompilerParams(needs_layout_passes=False),
        scratch_types=[
            pltpu.VMEM((2 * _C,), jnp.int32),
            pltpu.VMEM((_C,), jnp.int32),
            pltpu.VMEM((_C,), jnp.int32),
            pltpu.VMEM((_C,), jnp.int32),
            pltpu.VMEM((_C,), jnp.int32),
            pltpu.VMEM((_C, _D), jnp.float32),
            pltpu.VMEM((_C, _D), jnp.float32),
            pltpu.VMEM((_RC, _D), jnp.float32),
            pltpu.VMEM((8, _D), jnp.float32),
            pltpu.VMEM((_NPAD,), jnp.int32),
            pltpu.VMEM_SHARED((_ACCR, _D), jnp.float32),
            pltpu.SemaphoreType.DMA,
            pltpu.SemaphoreType.DMA,
        ],
    )
    def agg(x_hbm, edge_hbm, sum_hbm,
            pi_v, sia_v, dia_v, sib_v, dib_v, rowsa_v, rowsb_v, zr_v,
            c8_v, cnt_v, acc_sh, sema, semb):
        c = lax.axis_index("c")
        s = lax.axis_index("s")
        wid = s * _NC + c

        zeros16 = jnp.zeros((16,), jnp.float32)
        zeros16i = jnp.zeros((16,), jnp.int32)
        ones16i = jnp.ones((16,), jnp.int32)
        iota16 = lax.iota(jnp.int32, 16)

        def fill_zr(i, carry):
            for k in range(_D // 16):
                zr_v[i, pl.ds(k * 16, 16)] = zeros16
            return carry

        lax.fori_loop(0, _RC, fill_zr, 0)

        def fill_cnt(i, carry):
            cnt_v[pl.ds(i * 16, 16)] = zeros16i
            return carry

        lax.fori_loop(0, _NPAD // 16, fill_cnt, 0)

        base = s * _RPT

        def zinit(i, carry):
            pltpu.sync_copy(zr_v, acc_sh.at[pl.ds(base + i * _RC, _RC)])
            return carry

        lax.fori_loop(0, _RPT // _RC, zinit, 0)
        pltpu.sync_copy(zr_v.at[pl.ds(0, 8)],
                        acc_sh.at[pl.ds(_NPAD + 8 * s, 8)])
        plsc.subcore_barrier()

        def unpack(off, si_ref, di_ref):
            for t in range(_C // 16):
                w = pi_v[pl.ds(off + t * 16, 16)]
                si_ref[pl.ds(t * 16, 16)] = jnp.bitwise_and(w, 0xFFFF)
                d16 = lax.shift_right_logical(w, 16)
                di_ref[pl.ds(t * 16, 16)] = d16
                plsc.addupdate_scatter(cnt_v, [d16], ones16i)

        last = _NPAIR - 1

        def pair(gi, carry):
            pltpu.sync_copy(edge_hbm.at[wid, gi], pi_v)
            unpack(0, sia_v, dia_v)
            da = pltpu.async_copy(x_hbm.at[sia_v], rowsa_v, sema)

            @pl.when(gi < last)
            def _():
                unpack(_C, sib_v, dib_v)
                pltpu.async_copy(x_hbm.at[sib_v], rowsb_v, semb)

            @pl.when(gi == last)
            def _():
                for t in range(_C // 16):
                    dib_v[pl.ds(t * 16, 16)] = _NPAD + t * 16 + iota16

                def cfill(g, carry2):
                    v = cnt_v[pl.ds(g * 16, 16)].astype(jnp.float32)
                    row = g // 8
                    col = (g % 8) * 16
                    rowsb_v[row, pl.ds(col, 16)] = v
                    return carry2

                lax.fori_loop(0, _NPAD // 16, cfill, 0)

            da.wait()
            pltpu.sync_copy(rowsa_v, acc_sh.at[dia_v], add=True)

            @pl.when(gi < last)
            def _():
                pltpu.make_async_copy(x_hbm.at[pl.ds(0, _C)], rowsb_v,
                                      semb).wait()

            pltpu.sync_copy(rowsb_v, acc_sh.at[dib_v], add=True)
            return carry

        lax.fori_loop(0, _NPAIR, pair, 0)
        plsc.subcore_barrier()

        def wout(i, carry):
            off = base + i * _RC
            pltpu.sync_copy(acc_sh.at[pl.ds(off, _RC)], zr_v)
            pltpu.sync_copy(zr_v, sum_hbm.at[c, pl.ds(off, _RC)])
            return carry

        lax.fori_loop(0, _RPT // _RC, wout, 0)

        @pl.when(s < 10)
        def _():
            pltpu.sync_copy(acc_sh.at[pl.ds(_NPAD + 8 * s, 8)], c8_v)
            pltpu.sync_copy(c8_v, sum_hbm.at[c, pl.ds(_NPAD + 8 * s, 8)])

    return agg(x_t, edges)


def _dense(psum, pcnt, x_t, w_l, w_r, b, relu, logsm):
    br = 400

    def body(p_ref, c_ref, x_ref, wl_ref, wr_ref, b_ref, o_ref):
        p = p_ref[0] + p_ref[1]
        cnt = c_ref[0] + c_ref[1]
        mean = p / jnp.maximum(cnt, 1.0)
        out = (
            jnp.dot(mean, wl_ref[...], preferred_element_type=jnp.float32,
                    precision=lax.Precision.HIGHEST)
            + jnp.dot(x_ref[...], wr_ref[...],
                      preferred_element_type=jnp.float32,
                      precision=lax.Precision.HIGHEST)
            + b_ref[...]
        )
        if relu:
            out = jnp.maximum(out, 0.0)
        if logsm:
            m = jnp.max(out, axis=1, keepdims=True)
            out = out - m - jnp.log(
                jnp.sum(jnp.exp(out - m), axis=1, keepdims=True))
        o_ref[...] = out

    return pl.pallas_call(
        body,
        grid=(_N // br,),
        in_specs=[
            pl.BlockSpec((2, br, _D), lambda i: (0, i, 0)),
            pl.BlockSpec((2, br, 1), lambda i: (0, i, 0)),
            pl.BlockSpec((br, _D), lambda i: (i, 0)),
            pl.BlockSpec((_D, _D), lambda i: (0, 0)),
            pl.BlockSpec((_D, _D), lambda i: (0, 0)),
            pl.BlockSpec((1, _D), lambda i: (0, 0)),
        ],
        out_specs=pl.BlockSpec((br, _D), lambda i: (i, 0)),
        out_shape=jax.ShapeDtypeStruct((_N, _D), jnp.float32),
    )(psum, pcnt, x_t, w_l, w_r, b)


def _pack_edges(edge_index):
    packed = jnp.bitwise_or(edge_index[0],
                            jnp.left_shift(edge_index[1], 16))
    packed = packed.reshape(_NW, _EPW)
    pad = jnp.full((_NW, _NPAIR * 2 * _C - _EPW), _TRASH << 16,
                   dtype=jnp.int32)
    return jnp.concatenate([packed, pad], axis=1).reshape(
        _NW, _NPAIR, 2 * _C)


def kernel(x, edge_index1, edge_index2, W_l1, W_r1, b1, W_l2, W_r2, b2):
    e1 = _pack_edges(edge_index1)
    e2 = _pack_edges(edge_index2)

    s1 = _sc_agg(x, e1)
    c1 = s1[:, _NPAD:, :].reshape(_NC, _NPAD, 1)
    h = _dense(s1, c1, x, W_l1, W_r1, b1.reshape(1, _D), relu=True,
               logsm=False)
    s2 = _sc_agg(h, e2)
    c2 = s2[:, _NPAD:, :].reshape(_NC, _NPAD, 1)
    return _dense(s2, c2, h, W_l2, W_r2, b2.reshape(1, _D), relu=False,
                  logsm=True)


# speedup vs baseline: 8.5158x; 1.1380x over previous
import functools

import jax
import jax.numpy as jnp
from jax import lax
from jax.experimental import pallas as pl
from jax.experimental.pallas import tpu as pltpu
from jax.experimental.pallas import tpu_sc as plsc

_N = 10000
_E = 320000
_D = 128
_NC = 2
_NS = 16
_NW = _NC * _NS
_EPW = _E // _NW
_RC0 = 80
_C = 80
_NPAIR = 63
_NPAD = 10240
_ACCR = _NPAD + _RC0
_RPT = _NPAD // _NS
_RC = 80
_TRASH = 10200


def _sc_agg(x_t, edges):
    mesh = plsc.VectorSubcoreMesh(core_axis_name="c", subcore_axis_name="s")

    @functools.partial(
        pl.kernel,
        out_type=jax.ShapeDtypeStruct((_NC, _NPAD + _RC, _D), jnp.float32),
        mesh=mesh,
        compiler_params=pltpu.CompilerParams(needs_layout_passes=False),
        scratch_types=[
            pltpu.VMEM((2, 2 * _C), jnp.int32),
            pltpu.VMEM((_C,), jnp.int32),
            pltpu.VMEM((_C,), jnp.int32),
            pltpu.VMEM((_C,), jnp.int32),
            pltpu.VMEM((_C,), jnp.int32),
            pltpu.VMEM((_C, _D), jnp.float32),
            pltpu.VMEM((_C, _D), jnp.float32),
            pltpu.VMEM((_RC, _D), jnp.float32),
            pltpu.VMEM((8, _D), jnp.float32),
            pltpu.VMEM((_NPAD,), jnp.int32),
            pltpu.VMEM_SHARED((_ACCR, _D), jnp.float32),
            pltpu.SemaphoreType.DMA,
            pltpu.SemaphoreType.DMA,
            pltpu.SemaphoreType.DMA,
            pltpu.SemaphoreType.DMA,
        ],
    )
    def agg(x_hbm, edge_hbm, sum_hbm,
            pi_v, sia_v, dia_v, sib_v, dib_v, rowsa_v, rowsb_v, zr_v,
            c8_v, cnt_v, acc_sh, sema, semb, psem, ssem):
        c = lax.axis_index("c")
        s = lax.axis_index("s")
        wid = s * _NC + c

        zeros16 = jnp.zeros((16,), jnp.float32)
        zeros16i = jnp.zeros((16,), jnp.int32)
        ones16i = jnp.ones((16,), jnp.int32)
        iota16 = lax.iota(jnp.int32, 16)

        def fill_zr(i, carry):
            for k in range(_D // 16):
                zr_v[i, pl.ds(k * 16, 16)] = zeros16
            return carry

        lax.fori_loop(0, _RC, fill_zr, 0)

        def fill_cnt(i, carry):
            cnt_v[pl.ds(i * 16, 16)] = zeros16i
            return carry

        lax.fori_loop(0, _NPAD // 16, fill_cnt, 0)

        base = s * _RPT

        def zinit(i, carry):
            pltpu.sync_copy(zr_v, acc_sh.at[pl.ds(base + i * _RC, _RC)])
            return carry

        lax.fori_loop(0, _RPT // _RC, zinit, 0)
        pltpu.sync_copy(zr_v.at[pl.ds(0, 8)],
                        acc_sh.at[pl.ds(_NPAD + 8 * s, 8)])
        plsc.subcore_barrier()

        def unpack(r, off, si_ref, di_ref):
            for t in range(_C // 16):
                w = pi_v[r, pl.ds(off + t * 16, 16)]
                si_ref[pl.ds(t * 16, 16)] = jnp.bitwise_and(w, 0xFFFF)
                d16 = lax.shift_right_logical(w, 16)
                di_ref[pl.ds(t * 16, 16)] = d16
                plsc.addupdate_scatter(cnt_v, [d16], ones16i)

        last = _NPAIR - 1

        pltpu.async_copy(edge_hbm.at[wid, 0], pi_v.at[0], psem)

        def pair(gi, carry):
            r = gi % 2
            pltpu.make_async_copy(edge_hbm.at[wid, 0], pi_v.at[r],
                                  psem).wait()

            @pl.when(gi < last)
            def _():
                pltpu.async_copy(edge_hbm.at[wid, gi + 1],
                                 pi_v.at[1 - r], psem)

            @pl.when(gi > 0)
            def _():
                pltpu.make_async_copy(rowsb_v, acc_sh.at[pl.ds(0, _C)],
                                      ssem).wait()

            unpack(r, 0, sia_v, dia_v)
            da = pltpu.async_copy(x_hbm.at[sia_v], rowsa_v, sema)

            @pl.when(gi < last)
            def _():
                unpack(r, _C, sib_v, dib_v)
                pltpu.async_copy(x_hbm.at[sib_v], rowsb_v, semb)

            @pl.when(gi == last)
            def _():
                for t in range(_C // 16):
                    dib_v[pl.ds(t * 16, 16)] = _NPAD + t * 16 + iota16

                def cfill(g, carry2):
                    v = cnt_v[pl.ds(g * 16, 16)].astype(jnp.float32)
                    row = g // 8
                    col = (g % 8) * 16
                    rowsb_v[row, pl.ds(col, 16)] = v
                    return carry2

                lax.fori_loop(0, _NPAD // 16, cfill, 0)

            da.wait()
            pltpu.sync_copy(rowsa_v, acc_sh.at[dia_v], add=True)

            @pl.when(gi < last)
            def _():
                pltpu.make_async_copy(x_hbm.at[pl.ds(0, _C)], rowsb_v,
                                      semb).wait()

            pltpu.async_copy(rowsb_v, acc_sh.at[dib_v], ssem, add=True)
            return carry

        lax.fori_loop(0, _NPAIR, pair, 0)
        pltpu.make_async_copy(rowsb_v, acc_sh.at[pl.ds(0, _C)],
                              ssem).wait()
        plsc.subcore_barrier()

        def wout(i, carry):
            off = base + i * _RC
            pltpu.sync_copy(acc_sh.at[pl.ds(off, _RC)], zr_v)
            pltpu.sync_copy(zr_v, sum_hbm.at[c, pl.ds(off, _RC)])
            return carry

        lax.fori_loop(0, _RPT // _RC, wout, 0)

        @pl.when(s < 10)
        def _():
            pltpu.sync_copy(acc_sh.at[pl.ds(_NPAD + 8 * s, 8)], c8_v)
            pltpu.sync_copy(c8_v, sum_hbm.at[c, pl.ds(_NPAD + 8 * s, 8)])

    return agg(x_t, edges)


def _dense(psum, pcnt, x_t, w_l, w_r, b, relu, logsm):
    br = 400

    def body(p_ref, c_ref, x_ref, wl_ref, wr_ref, b_ref, o_ref):
        p = p_ref[0] + p_ref[1]
        cnt = c_ref[0] + c_ref[1]
        mean = p / jnp.maximum(cnt, 1.0)
        out = (
            jnp.dot(mean, wl_ref[...], preferred_element_type=jnp.float32,
                    precision=lax.Precision.HIGHEST)
            + jnp.dot(x_ref[...], wr_ref[...],
                      preferred_element_type=jnp.float32,
                      precision=lax.Precision.HIGHEST)
            + b_ref[...]
        )
        if relu:
            out = jnp.maximum(out, 0.0)
        if logsm:
            m = jnp.max(out, axis=1, keepdims=True)
            out = out - m - jnp.log(
                jnp.sum(jnp.exp(out - m), axis=1, keepdims=True))
        o_ref[...] = out

    return pl.pallas_call(
        body,
        grid=(_N // br,),
        in_specs=[
            pl.BlockSpec((2, br, _D), lambda i: (0, i, 0)),
            pl.BlockSpec((2, br, 1), lambda i: (0, i, 0)),
            pl.BlockSpec((br, _D), lambda i: (i, 0)),
            pl.BlockSpec((_D, _D), lambda i: (0, 0)),
            pl.BlockSpec((_D, _D), lambda i: (0, 0)),
            pl.BlockSpec((1, _D), lambda i: (0, 0)),
        ],
        out_specs=pl.BlockSpec((br, _D), lambda i: (i, 0)),
        out_shape=jax.ShapeDtypeStruct((_N, _D), jnp.float32),
    )(psum, pcnt, x_t, w_l, w_r, b)


def _pack_edges(edge_index):
    packed = jnp.bitwise_or(edge_index[0],
                            jnp.left_shift(edge_index[1], 16))
    packed = packed.reshape(_NW, _EPW)
    pad = jnp.full((_NW, _NPAIR * 2 * _C - _EPW), _TRASH << 16,
                   dtype=jnp.int32)
    return jnp.concatenate([packed, pad], axis=1).reshape(
        _NW, _NPAIR, 2 * _C)


def kernel(x, edge_index1, edge_index2, W_l1, W_r1, b1, W_l2, W_r2, b2):
    e1 = _pack_edges(edge_index1)
    e2 = _pack_edges(edge_index2)

    s1 = _sc_agg(x, e1)
    c1 = s1[:, _NPAD:, :].reshape(_NC, _NPAD, 1)
    h = _dense(s1, c1, x, W_l1, W_r1, b1.reshape(1, _D), relu=True,
               logsm=False)
    s2 = _sc_agg(h, e2)
    c2 = s2[:, _NPAD:, :].reshape(_NC, _NPAD, 1)
    return _dense(s2, c2, h, W_l2, W_r2, b2.reshape(1, _D), relu=False,
                  logsm=True)


# speedup vs baseline: 8.6600x; 1.0169x over previous
import functools

import jax
import jax.numpy as jnp
from jax import lax
from jax.experimental import pallas as pl
from jax.experimental.pallas import tpu as pltpu
from jax.experimental.pallas import tpu_sc as plsc

_N = 10000
_E = 320000
_D = 128
_NC = 2
_NS = 16
_NW = _NC * _NS
_EPW = _E // _NW
_RC0 = 80
_C = 80
_NPAIR = 63
_NPAD = 10240
_ACCR = _NPAD + _RC0
_RPT = _NPAD // _NS
_RC = 80
_TRASH = 10200


def _sc_agg(x_t, edges):
    mesh = plsc.VectorSubcoreMesh(core_axis_name="c", subcore_axis_name="s")

    @functools.partial(
        pl.kernel,
        out_type=jax.ShapeDtypeStruct((_NC, _NPAD + _RC, _D), jnp.float32),
        mesh=mesh,
        compiler_params=pltpu.CompilerParams(needs_layout_passes=False),
        scratch_types=[
            pltpu.VMEM((2, 2 * _C), jnp.int32),
            pltpu.VMEM((_C,), jnp.int32),
            pltpu.VMEM((_C,), jnp.int32),
            pltpu.VMEM((_C,), jnp.int32),
            pltpu.VMEM((_C,), jnp.int32),
            pltpu.VMEM((_C, _D), jnp.float32),
            pltpu.VMEM((_C, _D), jnp.float32),
            pltpu.VMEM((_RC, _D), jnp.float32),
            pltpu.VMEM((8, _D), jnp.float32),
            pltpu.VMEM((_NPAD,), jnp.int32),
            pltpu.VMEM_SHARED((_ACCR, _D), jnp.float32),
            pltpu.SemaphoreType.DMA,
            pltpu.SemaphoreType.DMA,
            pltpu.SemaphoreType.DMA,
            pltpu.SemaphoreType.DMA,
            pltpu.SemaphoreType.DMA,
        ],
    )
    def agg(x_hbm, edge_hbm, sum_hbm,
            pi_v, sia_v, dia_v, sib_v, dib_v, rowsa_v, rowsb_v, zr_v,
            c8_v, cnt_v, acc_sh, sema, semb, psem, ssem, asem):
        c = lax.axis_index("c")
        s = lax.axis_index("s")
        wid = s * _NC + c

        zeros16 = jnp.zeros((16,), jnp.float32)
        zeros16i = jnp.zeros((16,), jnp.int32)
        ones16i = jnp.ones((16,), jnp.int32)
        iota16 = lax.iota(jnp.int32, 16)

        def fill_zr(i, carry):
            for k in range(_D // 16):
                zr_v[i, pl.ds(k * 16, 16)] = zeros16
            return carry

        lax.fori_loop(0, _RC, fill_zr, 0)

        def fill_cnt(i, carry):
            cnt_v[pl.ds(i * 16, 16)] = zeros16i
            return carry

        lax.fori_loop(0, _NPAD // 16, fill_cnt, 0)

        base = s * _RPT

        def zinit(i, carry):
            pltpu.sync_copy(zr_v, acc_sh.at[pl.ds(base + i * _RC, _RC)])
            return carry

        lax.fori_loop(0, _RPT // _RC, zinit, 0)
        pltpu.sync_copy(zr_v.at[pl.ds(0, 8)],
                        acc_sh.at[pl.ds(_NPAD + 8 * s, 8)])
        plsc.subcore_barrier()

        def unpack(r, off, si_ref, di_ref):
            for t in range(_C // 16):
                w = pi_v[r, pl.ds(off + t * 16, 16)]
                si_ref[pl.ds(t * 16, 16)] = jnp.bitwise_and(w, 0xFFFF)
                d16 = lax.shift_right_logical(w, 16)
                di_ref[pl.ds(t * 16, 16)] = d16
                plsc.addupdate_scatter(cnt_v, [d16], ones16i)

        last = _NPAIR - 1

        pltpu.async_copy(edge_hbm.at[wid, 0], pi_v.at[0], psem)

        def pair(gi, carry):
            r = gi % 2
            pltpu.make_async_copy(edge_hbm.at[wid, 0], pi_v.at[r],
                                  psem).wait()

            @pl.when(gi < last)
            def _():
                pltpu.async_copy(edge_hbm.at[wid, gi + 1],
                                 pi_v.at[1 - r], psem)

            @pl.when(gi > 0)
            def _():
                pltpu.make_async_copy(rowsb_v, acc_sh.at[pl.ds(0, _C)],
                                      ssem).wait()
                pltpu.make_async_copy(rowsa_v, acc_sh.at[pl.ds(0, _C)],
                                      asem).wait()

            unpack(r, 0, sia_v, dia_v)
            da = pltpu.async_copy(x_hbm.at[sia_v], rowsa_v, sema)

            @pl.when(gi < last)
            def _():
                unpack(r, _C, sib_v, dib_v)
                pltpu.async_copy(x_hbm.at[sib_v], rowsb_v, semb)

            @pl.when(gi == last)
            def _():
                for t in range(_C // 16):
                    dib_v[pl.ds(t * 16, 16)] = _NPAD + t * 16 + iota16

                def cfill(g, carry2):
                    v = cnt_v[pl.ds(g * 16, 16)].astype(jnp.float32)
                    row = g // 8
                    col = (g % 8) * 16
                    rowsb_v[row, pl.ds(col, 16)] = v
                    return carry2

                lax.fori_loop(0, _NPAD // 16, cfill, 0)

            da.wait()
            pltpu.async_copy(rowsa_v, acc_sh.at[dia_v], asem, add=True)

            @pl.when(gi < last)
            def _():
                pltpu.make_async_copy(x_hbm.at[pl.ds(0, _C)], rowsb_v,
                                      semb).wait()

            pltpu.async_copy(rowsb_v, acc_sh.at[dib_v], ssem, add=True)
            return carry

        lax.fori_loop(0, _NPAIR, pair, 0)
        pltpu.make_async_copy(rowsb_v, acc_sh.at[pl.ds(0, _C)],
                              ssem).wait()
        pltpu.make_async_copy(rowsa_v, acc_sh.at[pl.ds(0, _C)],
                              asem).wait()
        plsc.subcore_barrier()

        def wout(i, carry):
            off = base + i * _RC
            pltpu.sync_copy(acc_sh.at[pl.ds(off, _RC)], zr_v)
            pltpu.sync_copy(zr_v, sum_hbm.at[c, pl.ds(off, _RC)])
            return carry

        lax.fori_loop(0, _RPT // _RC, wout, 0)

        @pl.when(s < 10)
        def _():
            pltpu.sync_copy(acc_sh.at[pl.ds(_NPAD + 8 * s, 8)], c8_v)
            pltpu.sync_copy(c8_v, sum_hbm.at[c, pl.ds(_NPAD + 8 * s, 8)])

    return agg(x_t, edges)


def _dense(psum, pcnt, x_t, w_l, w_r, b, relu, logsm):
    br = 400

    def body(p_ref, c_ref, x_ref, wl_ref, wr_ref, b_ref, o_ref):
        p = p_ref[0] + p_ref[1]
        cnt = c_ref[0] + c_ref[1]
        mean = p / jnp.maximum(cnt, 1.0)
        out = (
            jnp.dot(mean, wl_ref[...], preferred_element_type=jnp.float32,
                    precision=lax.Precision.HIGHEST)
            + jnp.dot(x_ref[...], wr_ref[...],
                      preferred_element_type=jnp.float32,
                      precision=lax.Precision.HIGHEST)
            + b_ref[...]
        )
        if relu:
            out = jnp.maximum(out, 0.0)
        if logsm:
            m = jnp.max(out, axis=1, keepdims=True)
            out = out - m - jnp.log(
                jnp.sum(jnp.exp(out - m), axis=1, keepdims=True))
        o_ref[...] = out

    return pl.pallas_call(
        body,
        grid=(_N // br,),
        in_specs=[
            pl.BlockSpec((2, br, _D), lambda i: (0, i, 0)),
            pl.BlockSpec((2, br, 1), lambda i: (0, i, 0)),
            pl.BlockSpec((br, _D), lambda i: (i, 0)),
            pl.BlockSpec((_D, _D), lambda i: (0, 0)),
            pl.BlockSpec((_D, _D), lambda i: (0, 0)),
            pl.BlockSpec((1, _D), lambda i: (0, 0)),
        ],
        out_specs=pl.BlockSpec((br, _D), lambda i: (i, 0)),
        out_shape=jax.ShapeDtypeStruct((_N, _D), jnp.float32),
    )(psum, pcnt, x_t, w_l, w_r, b)


def _pack_edges(edge_index):
    packed = jnp.bitwise_or(edge_index[0],
                            jnp.left_shift(edge_index[1], 16))
    packed = packed.reshape(_NW, _EPW)
    pad = jnp.full((_NW, _NPAIR * 2 * _C - _EPW), _TRASH << 16,
                   dtype=jnp.int32)
    return jnp.concatenate([packed, pad], axis=1).reshape(
        _NW, _NPAIR, 2 * _C)


def kernel(x, edge_index1, edge_index2, W_l1, W_r1, b1, W_l2, W_r2, b2):
    e1 = _pack_edges(edge_index1)
    e2 = _pack_edges(edge_index2)

    s1 = _sc_agg(x, e1)
    c1 = s1[:, _NPAD:, :].reshape(_NC, _NPAD, 1)
    h = _dense(s1, c1, x, W_l1, W_r1, b1.reshape(1, _D), relu=True,
               logsm=False)
    s2 = _sc_agg(h, e2)
    c2 = s2[:, _NPAD:, :].reshape(_NC, _NPAD, 1)
    return _dense(s2, c2, h, W_l2, W_r2, b2.reshape(1, _D), relu=False,
                  logsm=True)
